# baseline (speedup 1.0000x reference)
import jax
import jax.numpy as jnp
from jax import lax
from jax.experimental import pallas as pl
from jax.experimental.pallas import tpu as pltpu
from jax.experimental.pallas import tpu_sc as plsc

LEARN_RATE = 0.2
_NC, _NS, _L = 2, 16, 16
_NW = _NC * _NS
_NACT = 16


_CHUNK = 25000
_NBUF = 4


def _sc_body(table_ref, idx_ref, tgt_ref, act_ref, out_ref, loss_ref,
             idxv, actv, tgtv, fidxv, fidxm, qv, nqv, lossv,
             cbuf0, cbuf1, cbuf2, cbuf3, semc, semo, semg):
    cbuf = (cbuf0, cbuf1, cbuf2, cbuf3)
    tbl = table_ref.shape[0]
    half = tbl // _NC
    cslice = half // _NS
    nchunk = cslice // _CHUNK
    nch = idxv.shape[0]
    c = lax.axis_index("c")
    s = lax.axis_index("s")
    wid = s * _NC + c
    cbase = c * half + s * cslice
    ins = [None] * nchunk
    outs = [None] * nchunk
    for t in range(min(_NBUF, nchunk)):
        ins[t] = pltpu.async_copy(
            table_ref.at[pl.ds(cbase + t * _CHUNK, _CHUNK)],
            cbuf[t % _NBUF], semc)
    base = s * nch
    pltpu.sync_copy(idx_ref.at[pl.ds(base, nch)], idxv)
    pltpu.sync_copy(act_ref.at[pl.ds(base, nch)], actv)
    pltpu.sync_copy(tgt_ref.at[pl.ds(base, nch)], tgtv)
    lo = c * half
    for j in range(nch):
        for k in range(128 // _L):
            sl = pl.ds(k * _L, _L)
            f = idxv[j, sl] * _NACT + actv[j, sl]
            fidxv[j, sl] = f
            mine = (f >= lo) & (f < lo + half)
            fidxm[j, sl] = jnp.where(mine, f, -1)
    gathers = [
        pltpu.async_copy(table_ref.at[fidxv.at[j]], qv.at[j], semg)
        for j in range(nch)
    ]
    for t in range(nchunk):
        b = t % _NBUF
        ins[t].wait()
        outs[t] = pltpu.async_copy(
            cbuf[b], out_ref.at[pl.ds(cbase + t * _CHUNK, _CHUNK)], semo)
        nt = t + _NBUF
        if nt < nchunk:
            outs[t].wait()
            ins[nt] = pltpu.async_copy(
                table_ref.at[pl.ds(cbase + nt * _CHUNK, _CHUNK)],
                cbuf[b], semc)
    for g in gathers:
        g.wait()
    acc = jnp.zeros((_L,), jnp.float32)
    for j in range(nch):
        for k in range(128 // _L):
            sl = pl.ds(k * _L, _L)
            q = qv[j, sl]
            d = tgtv[j, sl] - q
            nqv[j, sl] = q + LEARN_RATE * d
            acc = acc + jnp.where(fidxm[j, sl] >= 0, d * d, 0.0)
    lossv[...] = acc
    for t in range(max(0, nchunk - _NBUF), nchunk):
        outs[t].wait()
    plsc.subcore_barrier()
    scatters = [
        pltpu.async_copy(
            nqv.at[j],
            out_ref.at[plsc.Indices(fidxm.at[j], ignored_value=-1)], semg)
        for j in range(nch)
    ]
    for sc in scatters:
        sc.wait()
    pltpu.sync_copy(lossv, loss_ref.at[wid])


def kernel(table, idx, targets, actions):
    batch = idx.shape[0]
    tflat = table.reshape(-1)
    rows = batch // 128
    nch = rows // _NS
    idx2 = idx.reshape(rows, 128)
    act2 = actions.reshape(rows, 128)
    tgt2 = targets.reshape(rows, 128)

    mesh = plsc.VectorSubcoreMesh(
        core_axis_name="c", subcore_axis_name="s",
        num_cores=_NC, num_subcores=_NS)
    sck = pl.kernel(
        _sc_body,
        out_type=(
            jax.ShapeDtypeStruct(tflat.shape, jnp.float32),
            jax.ShapeDtypeStruct((_NW, _L), jnp.float32),
        ),
        mesh=mesh,
        scratch_types=[
            pltpu.VMEM((nch, 128), jnp.int32),
            pltpu.VMEM((nch, 128), jnp.int32),
            pltpu.VMEM((nch, 128), jnp.float32),
            pltpu.VMEM((nch, 128), jnp.int32),
            pltpu.VMEM((nch, 128), jnp.int32),
            pltpu.VMEM((nch, 128), jnp.float32),
            pltpu.VMEM((nch, 128), jnp.float32),
            pltpu.VMEM((_L,), jnp.float32),
            pltpu.VMEM((_CHUNK,), jnp.float32),
            pltpu.VMEM((_CHUNK,), jnp.float32),
            pltpu.VMEM((_CHUNK,), jnp.float32),
            pltpu.VMEM((_CHUNK,), jnp.float32),
            pltpu.SemaphoreType.DMA,
            pltpu.SemaphoreType.DMA,
            pltpu.SemaphoreType.DMA,
        ],
    )
    newtab, loss_part = sck(tflat, idx2, tgt2, act2)
    return newtab.reshape(table.shape), jnp.sum(loss_part)
